# baseline (speedup 1.0000x reference)
import jax
import jax.numpy as jnp
from jax.experimental import pallas as pl
from jax.experimental.pallas import tpu as pltpu

_B, _S, _HID = 1, 2048, 2048
_NH, _NKV, _HD = 16, 8, 128
_ROPE_THETA = 10000.0
_SINK, _LOCAL = 64, 256
_NRET = 8
_NRKV = _NRET // 2
_REP = _NH // _NKV
_QKV_W = (_NH + 2 * _NKV) * _HD
_QK_W = (_NH + _NKV) * _HD
_NQK = _NH + _NKV
_BQ = 256
_NQ = _S // _BQ
_BQA = 512
_BQ2 = 2 * _BQA
_NQA = _S // _BQA
_SCALE = _HD ** -0.5
_NEG = -1e30
_PART = 512
_NPART = _S // _PART


def _qkv_rope_kernel(x_ref, w_ref, ct_ref, st_ref, o_ref):
    acc = jnp.dot(x_ref[...].astype(jnp.bfloat16), w_ref[...],
                  preferred_element_type=jnp.float32)
    qk = acc[:, :_QK_W]
    pieces = []
    for c in range(_NQK):
        x1 = qk[:, c * _HD: c * _HD + _HD // 2]
        x2 = qk[:, c * _HD + _HD // 2: (c + 1) * _HD]
        pieces += [-x2, x1]
    rot = jnp.concatenate(pieces, axis=1)
    ct = jnp.concatenate([ct_ref[...]] * _NQK, axis=1)
    st = jnp.concatenate([st_ref[...]] * _NQK, axis=1)
    roped = qk * ct + rot * st
    o_ref[...] = jnp.concatenate(
        [roped, acc[:, _QK_W:]], axis=1).astype(jnp.bfloat16)


def _out_proj_kernel(x_ref, w_ref, o_ref):
    o_ref[...] = jnp.dot(x_ref[...], w_ref[...],
                         preferred_element_type=jnp.float32)


def _partial(q2, k_ref, v_ref, start, qpos, masked):
    k = k_ref[pl.ds(start, _PART), :]
    v = v_ref[pl.ds(start, _PART), :]
    s = jax.lax.dot_general(
        q2, k, (((1,), (1,)), ((), ())),
        preferred_element_type=jnp.float32)
    if masked:
        kpos = start + jax.lax.broadcasted_iota(jnp.int32, (_BQ2, _PART), 1)
        s = jnp.where(kpos <= qpos, s, _NEG)
    m = jnp.max(s, axis=1, keepdims=True)
    p = jnp.exp((s - m) * _SCALE)
    l = jnp.sum(p, axis=1, keepdims=True)
    o = jnp.dot(p.astype(jnp.bfloat16), v,
                preferred_element_type=jnp.float32)
    return m, l, o


_WIN = 768


def _attn_kernel(q_ref, k_ref, v_ref, o_ref, m_s, l_s, o_s):
    kh = pl.program_id(0)
    qb = pl.program_id(1)
    qblk = q_ref[...]
    q2 = jnp.concatenate([qblk[:, :_HD], qblk[:, _HD:]], axis=0)
    qpos1 = qb * _BQA + jax.lax.broadcasted_iota(jnp.int32, (_BQA, 1), 0)
    qpos = jnp.concatenate([qpos1, qpos1], axis=0)

    @pl.when(kh < _NRKV)
    def _retrieval():
        def _init(masked):
            m0, l0, o0 = _partial(q2, k_ref, v_ref, 0, qpos, masked)
            m_s[...] = jnp.broadcast_to(m0, (_BQ2, _HD))
            l_s[...] = jnp.broadcast_to(l0, (_BQ2, _HD))
            o_s[...] = o0

        def _accum(i, masked):
            mi, li, oi = _partial(q2, k_ref, v_ref, i * _PART, qpos, masked)
            m_prev = m_s[...]
            m_new = jnp.maximum(m_prev, mi)
            a = jnp.exp((m_prev - m_new) * _SCALE)
            b = jnp.exp((mi - m_new) * _SCALE)
            l_s[...] = a * l_s[...] + b * li
            o_s[...] = a * o_s[...] + b * oi
            m_s[...] = m_new

        pl.when(qb == 0)(lambda: _init(True))
        pl.when(qb > 0)(lambda: _init(False))
        for i in range(1, _NPART):
            pl.when(qb == i)(lambda i=i: _accum(i, True))
            pl.when(qb > i)(lambda i=i: _accum(i, False))

    @pl.when(kh >= _NRKV)
    def _streaming():
        wstart = jnp.maximum(qb * 2 - 1, 0) * 256
        kw = k_ref[pl.ds(wstart, _WIN), :]
        vw = v_ref[pl.ds(wstart, _WIN), :]
        k0 = k_ref[0:256, :]
        v0 = v_ref[0:256, :]
        sw = jax.lax.dot_general(
            q2, kw, (((1,), (1,)), ((), ())),
            preferred_element_type=jnp.float32)
        s0 = jax.lax.dot_general(
            q2, k0, (((1,), (1,)), ((), ())),
            preferred_element_type=jnp.float32)
        kw_pos = wstart + jax.lax.broadcasted_iota(
            jnp.int32, (_BQ2, _WIN), 1)
        mask_w = (kw_pos <= qpos) & ((kw_pos < _SINK)
                                     | ((qpos - kw_pos) < _LOCAL))
        k0_pos = jax.lax.broadcasted_iota(jnp.int32, (_BQ2, 256), 1)
        mask_0 = (k0_pos < _SINK) & (k0_pos < wstart)
        sw = jnp.where(mask_w, sw, _NEG)
        s0 = jnp.where(mask_0, s0, _NEG)
        m = jnp.maximum(jnp.max(sw, axis=1, keepdims=True),
                        jnp.max(s0, axis=1, keepdims=True))
        pw = jnp.exp((sw - m) * _SCALE)
        p0 = jnp.exp((s0 - m) * _SCALE)
        l = (jnp.sum(pw, axis=1, keepdims=True)
             + jnp.sum(p0, axis=1, keepdims=True))
        o = (jnp.dot(pw.astype(jnp.bfloat16), vw,
                     preferred_element_type=jnp.float32)
             + jnp.dot(p0.astype(jnp.bfloat16), v0,
                       preferred_element_type=jnp.float32))
        m_s[...] = jnp.broadcast_to(m, (_BQ2, _HD))
        l_s[...] = jnp.broadcast_to(l, (_BQ2, _HD))
        o_s[...] = o

    o2 = o_s[...] / l_s[...]
    o_ref[...] = jnp.concatenate(
        [o2[:_BQA, :], o2[_BQA:, :]], axis=1).astype(jnp.bfloat16)


def kernel(hidden_states, wqkv, wo):
    hs = hidden_states.reshape(_S, _HID)
    wq = wqkv.astype(jnp.bfloat16)
    wo_b = wo.astype(jnp.bfloat16)

    half = _HD // 2
    inv_freq = 1.0 / (_ROPE_THETA ** (
        jnp.arange(0, half, dtype=jnp.float32) / half))
    pos = jnp.arange(_S, dtype=jnp.float32)
    angles = pos[:, None] * inv_freq[None, :]
    ctab = jnp.concatenate([jnp.cos(angles)] * 2, axis=-1)
    stab = jnp.concatenate([jnp.sin(angles)] * 2, axis=-1)

    qkv = pl.pallas_call(
        _qkv_rope_kernel,
        grid=(_NQ,),
        in_specs=[
            pl.BlockSpec((_BQ, _HID), lambda i: (i, 0)),
            pl.BlockSpec((_HID, _QKV_W), lambda i: (0, 0)),
            pl.BlockSpec((_BQ, _HD), lambda i: (i, 0)),
            pl.BlockSpec((_BQ, _HD), lambda i: (i, 0)),
        ],
        out_specs=pl.BlockSpec((_BQ, _QKV_W), lambda i: (i, 0)),
        out_shape=jax.ShapeDtypeStruct((_S, _QKV_W), jnp.bfloat16),
    )(hs, wq, ctab, stab)

    attn = pl.pallas_call(
        _attn_kernel,
        grid=(_NKV, _NQA),
        in_specs=[
            pl.BlockSpec((_BQA, 2 * _HD), lambda kh, qb: (qb, kh)),
            pl.BlockSpec((_S, _HD), lambda kh, qb: (0, _NH + kh)),
            pl.BlockSpec((_S, _HD), lambda kh, qb: (0, _NH + _NKV + kh)),
        ],
        out_specs=pl.BlockSpec((_BQA, 2 * _HD), lambda kh, qb: (qb, kh)),
        out_shape=jax.ShapeDtypeStruct((_S, _NH * _HD), jnp.bfloat16),
        scratch_shapes=[
            pltpu.VMEM((_BQ2, _HD), jnp.float32),
            pltpu.VMEM((_BQ2, _HD), jnp.float32),
            pltpu.VMEM((_BQ2, _HD), jnp.float32),
        ],
    )(qkv, qkv, qkv)

    out = pl.pallas_call(
        _out_proj_kernel,
        grid=(_NQ,),
        in_specs=[
            pl.BlockSpec((_BQ, _NH * _HD), lambda i: (i, 0)),
            pl.BlockSpec((_NH * _HD, _HID), lambda i: (0, 0)),
        ],
        out_specs=pl.BlockSpec((_BQ, _HID), lambda i: (i, 0)),
        out_shape=jax.ShapeDtypeStruct((_S, _HID), jnp.float32),
    )(attn, wo_b)

    return out.reshape(_B, _S, _HID)


# speedup vs baseline: 1.0204x; 1.0204x over previous
import jax
import jax.numpy as jnp
from jax.experimental import pallas as pl
from jax.experimental.pallas import tpu as pltpu

_B, _S, _HID = 1, 2048, 2048
_NH, _NKV, _HD = 16, 8, 128
_ROPE_THETA = 10000.0
_SINK, _LOCAL = 64, 256
_NRET = 8
_NRKV = _NRET // 2
_REP = _NH // _NKV
_QKV_W = (_NH + 2 * _NKV) * _HD
_QK_W = (_NH + _NKV) * _HD
_NQK = _NH + _NKV
_BQ = 256
_NQ = _S // _BQ
_BQA = 512
_BQ2 = 2 * _BQA
_NQA = _S // _BQA
_SCALE = _HD ** -0.5
_NEG = -1e30
_PART = 512
_NPART = _S // _PART


def _qkv_rope_kernel(x_ref, w_ref, ct_ref, st_ref, o_ref):
    acc = jnp.dot(x_ref[...].astype(jnp.bfloat16), w_ref[...],
                  preferred_element_type=jnp.float32)
    qk = acc[:, :_QK_W]
    pieces = []
    for c in range(_NQK):
        x1 = qk[:, c * _HD: c * _HD + _HD // 2]
        x2 = qk[:, c * _HD + _HD // 2: (c + 1) * _HD]
        pieces += [-x2, x1]
    rot = jnp.concatenate(pieces, axis=1)
    ct = jnp.concatenate([ct_ref[...]] * _NQK, axis=1)
    st = jnp.concatenate([st_ref[...]] * _NQK, axis=1)
    roped = qk * ct + rot * st
    o_ref[...] = jnp.concatenate(
        [roped, acc[:, _QK_W:]], axis=1).astype(jnp.bfloat16)


def _out_proj_kernel(x_ref, w_ref, o_ref, wb_s):
    @pl.when(pl.program_id(0) == 0)
    def _cast():
        wb_s[...] = w_ref[...].astype(jnp.bfloat16)

    o_ref[...] = jnp.dot(x_ref[...], wb_s[...],
                         preferred_element_type=jnp.float32)


def _partial(q2, k_ref, v_ref, start, qpos, masked):
    k = k_ref[pl.ds(start, _PART), :]
    v = v_ref[pl.ds(start, _PART), :]
    s = jax.lax.dot_general(
        q2, k, (((1,), (1,)), ((), ())),
        preferred_element_type=jnp.float32)
    if masked:
        kpos = start + jax.lax.broadcasted_iota(jnp.int32, (_BQ2, _PART), 1)
        s = jnp.where(kpos <= qpos, s, _NEG)
    m = jnp.max(s, axis=1, keepdims=True)
    p = jnp.exp((s - m) * _SCALE)
    l = jnp.sum(p, axis=1, keepdims=True)
    o = jnp.dot(p.astype(jnp.bfloat16), v,
                preferred_element_type=jnp.float32)
    return m, l, o


_WIN = 768


def _attn_kernel(q_ref, k_ref, v_ref, o_ref, m_s, l_s, o_s):
    kh = pl.program_id(0)
    qb = pl.program_id(1)
    qblk = q_ref[...]
    q2 = jnp.concatenate([qblk[:, :_HD], qblk[:, _HD:]], axis=0)
    qpos1 = qb * _BQA + jax.lax.broadcasted_iota(jnp.int32, (_BQA, 1), 0)
    qpos = jnp.concatenate([qpos1, qpos1], axis=0)

    @pl.when(kh < _NRKV)
    def _retrieval():
        def _init(masked):
            m0, l0, o0 = _partial(q2, k_ref, v_ref, 0, qpos, masked)
            m_s[...] = jnp.broadcast_to(m0, (_BQ2, _HD))
            l_s[...] = jnp.broadcast_to(l0, (_BQ2, _HD))
            o_s[...] = o0

        def _accum(i, masked):
            mi, li, oi = _partial(q2, k_ref, v_ref, i * _PART, qpos, masked)
            m_prev = m_s[...]
            m_new = jnp.maximum(m_prev, mi)
            a = jnp.exp((m_prev - m_new) * _SCALE)
            b = jnp.exp((mi - m_new) * _SCALE)
            l_s[...] = a * l_s[...] + b * li
            o_s[...] = a * o_s[...] + b * oi
            m_s[...] = m_new

        _init(True)
        for i in range(1, _NPART):
            pl.when(qb >= i)(lambda i=i: _accum(i, True))

    @pl.when(kh >= _NRKV)
    def _streaming():
        wstart = jnp.maximum(qb * 2 - 1, 0) * 256
        kw = k_ref[pl.ds(wstart, _WIN), :]
        vw = v_ref[pl.ds(wstart, _WIN), :]
        k0 = k_ref[0:256, :]
        v0 = v_ref[0:256, :]
        sw = jax.lax.dot_general(
            q2, kw, (((1,), (1,)), ((), ())),
            preferred_element_type=jnp.float32)
        s0 = jax.lax.dot_general(
            q2, k0, (((1,), (1,)), ((), ())),
            preferred_element_type=jnp.float32)
        kw_pos = wstart + jax.lax.broadcasted_iota(
            jnp.int32, (_BQ2, _WIN), 1)
        mask_w = (kw_pos <= qpos) & ((kw_pos < _SINK)
                                     | ((qpos - kw_pos) < _LOCAL))
        k0_pos = jax.lax.broadcasted_iota(jnp.int32, (_BQ2, 256), 1)
        mask_0 = (k0_pos < _SINK) & (k0_pos < wstart)
        sw = jnp.where(mask_w, sw, _NEG)
        s0 = jnp.where(mask_0, s0, _NEG)
        m = jnp.maximum(jnp.max(sw, axis=1, keepdims=True),
                        jnp.max(s0, axis=1, keepdims=True))
        pw = jnp.exp((sw - m) * _SCALE)
        p0 = jnp.exp((s0 - m) * _SCALE)
        l = (jnp.sum(pw, axis=1, keepdims=True)
             + jnp.sum(p0, axis=1, keepdims=True))
        o = (jnp.dot(pw.astype(jnp.bfloat16), vw,
                     preferred_element_type=jnp.float32)
             + jnp.dot(p0.astype(jnp.bfloat16), v0,
                       preferred_element_type=jnp.float32))
        m_s[...] = jnp.broadcast_to(m, (_BQ2, _HD))
        l_s[...] = jnp.broadcast_to(l, (_BQ2, _HD))
        o_s[...] = o

    o2 = o_s[...] / l_s[...]
    o_ref[...] = jnp.concatenate(
        [o2[:_BQA, :], o2[_BQA:, :]], axis=1).astype(jnp.bfloat16)


def kernel(hidden_states, wqkv, wo):
    hs = hidden_states.reshape(_S, _HID)
    wq = wqkv.astype(jnp.bfloat16)

    half = _HD // 2
    inv_freq = 1.0 / (_ROPE_THETA ** (
        jnp.arange(0, half, dtype=jnp.float32) / half))
    pos = jnp.arange(_S, dtype=jnp.float32)
    angles = pos[:, None] * inv_freq[None, :]
    ctab = jnp.concatenate([jnp.cos(angles)] * 2, axis=-1)
    stab = jnp.concatenate([jnp.sin(angles)] * 2, axis=-1)

    qkv = pl.pallas_call(
        _qkv_rope_kernel,
        grid=(_NQ,),
        in_specs=[
            pl.BlockSpec((_BQ, _HID), lambda i: (i, 0)),
            pl.BlockSpec((_HID, _QKV_W), lambda i: (0, 0)),
            pl.BlockSpec((_BQ, _HD), lambda i: (i, 0)),
            pl.BlockSpec((_BQ, _HD), lambda i: (i, 0)),
        ],
        out_specs=pl.BlockSpec((_BQ, _QKV_W), lambda i: (i, 0)),
        out_shape=jax.ShapeDtypeStruct((_S, _QKV_W), jnp.bfloat16),
    )(hs, wq, ctab, stab)

    attn = pl.pallas_call(
        _attn_kernel,
        grid=(_NKV, _NQA),
        in_specs=[
            pl.BlockSpec((_BQA, 2 * _HD), lambda kh, qb: (qb, kh)),
            pl.BlockSpec((_S, _HD), lambda kh, qb: (0, _NH + kh)),
            pl.BlockSpec((_S, _HD), lambda kh, qb: (0, _NH + _NKV + kh)),
        ],
        out_specs=pl.BlockSpec((_BQA, 2 * _HD), lambda kh, qb: (qb, kh)),
        out_shape=jax.ShapeDtypeStruct((_S, _NH * _HD), jnp.bfloat16),
        scratch_shapes=[
            pltpu.VMEM((_BQ2, _HD), jnp.float32),
            pltpu.VMEM((_BQ2, _HD), jnp.float32),
            pltpu.VMEM((_BQ2, _HD), jnp.float32),
        ],
    )(qkv, qkv, qkv)

    out = pl.pallas_call(
        _out_proj_kernel,
        grid=(_NQ,),
        in_specs=[
            pl.BlockSpec((_BQ, _NH * _HD), lambda i: (i, 0)),
            pl.BlockSpec((_NH * _HD, _HID), lambda i: (0, 0)),
        ],
        out_specs=pl.BlockSpec((_BQ, _HID), lambda i: (i, 0)),
        out_shape=jax.ShapeDtypeStruct((_S, _HID), jnp.float32),
        scratch_shapes=[pltpu.VMEM((_NH * _HD, _HID), jnp.bfloat16)],
    )(attn, wo)

    return out.reshape(_B, _S, _HID)


# speedup vs baseline: 1.0776x; 1.0560x over previous
import jax
import jax.numpy as jnp
from jax.experimental import pallas as pl
from jax.experimental.pallas import tpu as pltpu

_B, _S, _HID = 1, 2048, 2048
_NH, _NKV, _HD = 16, 8, 128
_ROPE_THETA = 10000.0
_SINK, _LOCAL = 64, 256
_NRET = 8
_NRKV = _NRET // 2
_REP = _NH // _NKV
_QKV_W = (_NH + 2 * _NKV) * _HD
_QK_W = (_NH + _NKV) * _HD
_NQK = _NH + _NKV
_BQ = 256
_NQ = _S // _BQ
_BQA = 512
_BQ2 = 2 * _BQA
_NQA = _S // _BQA
_SCALE = _HD ** -0.5
_NEG = -1e30
_PART = 512
_NPART = _S // _PART


def _qkv_rope_kernel(x_ref, w_ref, ct_ref, st_ref, o_ref, wb_s):
    @pl.when(pl.program_id(0) == 0)
    def _cast():
        wb_s[...] = w_ref[...].astype(jnp.bfloat16)

    acc = jnp.dot(x_ref[...].astype(jnp.bfloat16), wb_s[...],
                  preferred_element_type=jnp.float32)
    qk = acc[:, :_QK_W]
    pieces = []
    for c in range(_NQK):
        x1 = qk[:, c * _HD: c * _HD + _HD // 2]
        x2 = qk[:, c * _HD + _HD // 2: (c + 1) * _HD]
        pieces += [-x2, x1]
    rot = jnp.concatenate(pieces, axis=1)
    ct = jnp.concatenate([ct_ref[...]] * _NQK, axis=1)
    st = jnp.concatenate([st_ref[...]] * _NQK, axis=1)
    roped = qk * ct + rot * st
    o_ref[...] = jnp.concatenate(
        [roped, acc[:, _QK_W:]], axis=1).astype(jnp.bfloat16)


def _out_proj_kernel(x_ref, w_ref, o_ref, wb_s):
    @pl.when(pl.program_id(0) == 0)
    def _cast():
        wb_s[...] = w_ref[...].astype(jnp.bfloat16)

    o_ref[...] = jnp.dot(x_ref[...], wb_s[...],
                         preferred_element_type=jnp.float32)


def _partial(q2, k_ref, v_ref, start, qpos, masked):
    k = k_ref[pl.ds(start, _PART), :]
    v = v_ref[pl.ds(start, _PART), :]
    s = jax.lax.dot_general(
        q2, k, (((1,), (1,)), ((), ())),
        preferred_element_type=jnp.float32)
    if masked:
        kpos = start + jax.lax.broadcasted_iota(jnp.int32, (_BQ2, _PART), 1)
        s = jnp.where(kpos <= qpos, s, _NEG)
    m = jnp.max(s, axis=1, keepdims=True)
    p = jnp.exp((s - m) * _SCALE)
    l = jnp.sum(p, axis=1, keepdims=True)
    o = jnp.dot(p.astype(jnp.bfloat16), v,
                preferred_element_type=jnp.float32)
    return m, l, o


_WIN = 768


def _attn_kernel(q_ref, k_ref, v_ref, o_ref, m_s, l_s, o_s):
    kh = pl.program_id(0)
    qb = pl.program_id(1)
    qblk = q_ref[...]
    q2 = jnp.concatenate([qblk[:, :_HD], qblk[:, _HD:]], axis=0)
    qpos1 = qb * _BQA + jax.lax.broadcasted_iota(jnp.int32, (_BQA, 1), 0)
    qpos = jnp.concatenate([qpos1, qpos1], axis=0)

    @pl.when(kh < _NRKV)
    def _retrieval():
        def _init(masked):
            m0, l0, o0 = _partial(q2, k_ref, v_ref, 0, qpos, masked)
            m_s[...] = jnp.broadcast_to(m0, (_BQ2, _HD))
            l_s[...] = jnp.broadcast_to(l0, (_BQ2, _HD))
            o_s[...] = o0

        def _accum(i, masked):
            mi, li, oi = _partial(q2, k_ref, v_ref, i * _PART, qpos, masked)
            m_prev = m_s[...]
            m_new = jnp.maximum(m_prev, mi)
            a = jnp.exp((m_prev - m_new) * _SCALE)
            b = jnp.exp((mi - m_new) * _SCALE)
            l_s[...] = a * l_s[...] + b * li
            o_s[...] = a * o_s[...] + b * oi
            m_s[...] = m_new

        _init(True)
        for i in range(1, _NPART):
            pl.when(qb >= i)(lambda i=i: _accum(i, True))

    @pl.when(kh >= _NRKV)
    def _streaming():
        wstart = jnp.maximum(qb * 2 - 1, 0) * 256
        kw = k_ref[pl.ds(wstart, _WIN), :]
        vw = v_ref[pl.ds(wstart, _WIN), :]
        k0 = k_ref[0:256, :]
        v0 = v_ref[0:256, :]
        sw = jax.lax.dot_general(
            q2, kw, (((1,), (1,)), ((), ())),
            preferred_element_type=jnp.float32)
        s0 = jax.lax.dot_general(
            q2, k0, (((1,), (1,)), ((), ())),
            preferred_element_type=jnp.float32)
        kw_pos = wstart + jax.lax.broadcasted_iota(
            jnp.int32, (_BQ2, _WIN), 1)
        mask_w = (kw_pos <= qpos) & ((kw_pos < _SINK)
                                     | ((qpos - kw_pos) < _LOCAL))
        k0_pos = jax.lax.broadcasted_iota(jnp.int32, (_BQ2, 256), 1)
        mask_0 = (k0_pos < _SINK) & (k0_pos < wstart)
        sw = jnp.where(mask_w, sw, _NEG)
        s0 = jnp.where(mask_0, s0, _NEG)
        m = jnp.maximum(jnp.max(sw, axis=1, keepdims=True),
                        jnp.max(s0, axis=1, keepdims=True))
        pw = jnp.exp((sw - m) * _SCALE)
        p0 = jnp.exp((s0 - m) * _SCALE)
        l = (jnp.sum(pw, axis=1, keepdims=True)
             + jnp.sum(p0, axis=1, keepdims=True))
        o = (jnp.dot(pw.astype(jnp.bfloat16), vw,
                     preferred_element_type=jnp.float32)
             + jnp.dot(p0.astype(jnp.bfloat16), v0,
                       preferred_element_type=jnp.float32))
        m_s[...] = jnp.broadcast_to(m, (_BQ2, _HD))
        l_s[...] = jnp.broadcast_to(l, (_BQ2, _HD))
        o_s[...] = o

    o2 = o_s[...] / l_s[...]
    o_ref[...] = jnp.concatenate(
        [o2[:_BQA, :], o2[_BQA:, :]], axis=1).astype(jnp.bfloat16)


def kernel(hidden_states, wqkv, wo):
    hs = hidden_states.reshape(_S, _HID)

    half = _HD // 2
    inv_freq = 1.0 / (_ROPE_THETA ** (
        jnp.arange(0, half, dtype=jnp.float32) / half))
    pos = jnp.arange(_S, dtype=jnp.float32)
    angles = pos[:, None] * inv_freq[None, :]
    ctab = jnp.concatenate([jnp.cos(angles)] * 2, axis=-1)
    stab = jnp.concatenate([jnp.sin(angles)] * 2, axis=-1)

    qkv = pl.pallas_call(
        _qkv_rope_kernel,
        grid=(_NQ,),
        in_specs=[
            pl.BlockSpec((_BQ, _HID), lambda i: (i, 0)),
            pl.BlockSpec((_HID, _QKV_W), lambda i: (0, 0)),
            pl.BlockSpec((_BQ, _HD), lambda i: (i, 0)),
            pl.BlockSpec((_BQ, _HD), lambda i: (i, 0)),
        ],
        out_specs=pl.BlockSpec((_BQ, _QKV_W), lambda i: (i, 0)),
        out_shape=jax.ShapeDtypeStruct((_S, _QKV_W), jnp.bfloat16),
        scratch_shapes=[pltpu.VMEM((_HID, _QKV_W), jnp.bfloat16)],
    )(hs, wqkv, ctab, stab)

    attn = pl.pallas_call(
        _attn_kernel,
        grid=(_NKV, _NQA),
        in_specs=[
            pl.BlockSpec((_BQA, 2 * _HD), lambda kh, qb: (qb, kh)),
            pl.BlockSpec((_S, _HD), lambda kh, qb: (0, _NH + kh)),
            pl.BlockSpec((_S, _HD), lambda kh, qb: (0, _NH + _NKV + kh)),
        ],
        out_specs=pl.BlockSpec((_BQA, 2 * _HD), lambda kh, qb: (qb, kh)),
        out_shape=jax.ShapeDtypeStruct((_S, _NH * _HD), jnp.bfloat16),
        scratch_shapes=[
            pltpu.VMEM((_BQ2, _HD), jnp.float32),
            pltpu.VMEM((_BQ2, _HD), jnp.float32),
            pltpu.VMEM((_BQ2, _HD), jnp.float32),
        ],
    )(qkv, qkv, qkv)

    out = pl.pallas_call(
        _out_proj_kernel,
        grid=(_NQ,),
        in_specs=[
            pl.BlockSpec((_BQ, _NH * _HD), lambda i: (i, 0)),
            pl.BlockSpec((_NH * _HD, _HID), lambda i: (0, 0)),
        ],
        out_specs=pl.BlockSpec((_BQ, _HID), lambda i: (i, 0)),
        out_shape=jax.ShapeDtypeStruct((_S, _HID), jnp.float32),
        scratch_shapes=[pltpu.VMEM((_NH * _HD, _HID), jnp.bfloat16)],
    )(attn, wo)

    return out.reshape(_B, _S, _HID)


# speedup vs baseline: 1.1644x; 1.0806x over previous
import jax
import jax.numpy as jnp
from jax.experimental import pallas as pl
from jax.experimental.pallas import tpu as pltpu

_B, _S, _HID = 1, 2048, 2048
_NH, _NKV, _HD = 16, 8, 128
_ROPE_THETA = 10000.0
_SINK, _LOCAL = 64, 256
_NRET = 8
_NRKV = _NRET // 2
_REP = _NH // _NKV
_QKV_W = (_NH + 2 * _NKV) * _HD
_QK_W = (_NH + _NKV) * _HD
_NQK = _NH + _NKV
_BQ = 256
_NQ = _S // _BQ
_BQA = 512
_BQ2 = 2 * _BQA
_NQA = _S // _BQA
_SCALE = _HD ** -0.5
_NEG = -1e30
_PART = 512
_NPART = _S // _PART


def _qkv_rope_kernel(x_ref, w_ref, ct_ref, st_ref, o_ref, wb_s):
    @pl.when(pl.program_id(0) == 0)
    def _cast():
        wb_s[...] = w_ref[...].astype(jnp.bfloat16)

    acc = jnp.dot(x_ref[...].astype(jnp.bfloat16), wb_s[...],
                  preferred_element_type=jnp.float32)
    qk = acc[:, :_QK_W]
    pieces = []
    for c in range(_NQK):
        x1 = qk[:, c * _HD: c * _HD + _HD // 2]
        x2 = qk[:, c * _HD + _HD // 2: (c + 1) * _HD]
        pieces += [-x2, x1]
    rot = jnp.concatenate(pieces, axis=1)
    ct = jnp.concatenate([ct_ref[...]] * _NQK, axis=1)
    st = jnp.concatenate([st_ref[...]] * _NQK, axis=1)
    roped = qk * ct + rot * st
    o_ref[...] = jnp.concatenate(
        [roped, acc[:, _QK_W:]], axis=1).astype(jnp.bfloat16)


def _out_proj_kernel(x_ref, w_ref, o_ref, wb_s):
    @pl.when(pl.program_id(0) == 0)
    def _cast():
        wb_s[...] = w_ref[...].astype(jnp.bfloat16)

    o_ref[...] = jnp.dot(x_ref[...], wb_s[...],
                         preferred_element_type=jnp.float32)


def _partial(q2, k_ref, v_ref, start, qpos, masked):
    k = k_ref[pl.ds(start, _PART), :]
    v = v_ref[pl.ds(start, _PART), :]
    s = jax.lax.dot_general(
        q2, k, (((1,), (1,)), ((), ())),
        preferred_element_type=jnp.float32)
    if masked:
        kpos = start + jax.lax.broadcasted_iota(jnp.int32, (_BQ2, _PART), 1)
        s = jnp.where(kpos <= qpos, s, _NEG)
    m = jnp.max(s, axis=1, keepdims=True)
    p = jnp.exp((s - m) * _SCALE)
    v_ext = jnp.concatenate(
        [v, jnp.ones((_PART, _HD), jnp.bfloat16)], axis=1)
    o_ext = jnp.dot(p.astype(jnp.bfloat16), v_ext,
                    preferred_element_type=jnp.float32)
    return m, o_ext[:, _HD:], o_ext[:, :_HD]


_WIN = 768


def _attn_kernel(q_ref, k_ref, v_ref, o_ref, m_s, l_s, o_s):
    kh = pl.program_id(0)
    qb = pl.program_id(1)
    qblk = q_ref[...]
    q2 = jnp.concatenate([qblk[:, :_HD], qblk[:, _HD:]], axis=0)
    qpos1 = qb * _BQA + jax.lax.broadcasted_iota(jnp.int32, (_BQA, 1), 0)
    qpos = jnp.concatenate([qpos1, qpos1], axis=0)

    @pl.when(kh < _NRKV)
    def _retrieval():
        def _init(masked):
            m0, l0, o0 = _partial(q2, k_ref, v_ref, 0, qpos, masked)
            m_s[...] = jnp.broadcast_to(m0, (_BQ2, _HD))
            l_s[...] = l0
            o_s[...] = o0

        def _accum(i, masked):
            mi, li, oi = _partial(q2, k_ref, v_ref, i * _PART, qpos, masked)
            m_prev = m_s[...]
            m_new = jnp.maximum(m_prev, mi)
            a = jnp.exp((m_prev - m_new) * _SCALE)
            b = jnp.exp((mi - m_new) * _SCALE)
            l_s[...] = a * l_s[...] + b * li
            o_s[...] = a * o_s[...] + b * oi
            m_s[...] = m_new

        _init(True)
        for i in range(1, _NPART):
            pl.when(qb >= i)(lambda i=i: _accum(i, True))

    @pl.when(kh >= _NRKV)
    def _streaming():
        wstart = jnp.maximum(qb * 2 - 1, 0) * 256
        kw = k_ref[pl.ds(wstart, _WIN), :]
        vw = v_ref[pl.ds(wstart, _WIN), :]
        k0 = k_ref[0:256, :]
        v0 = v_ref[0:256, :]
        sw = jax.lax.dot_general(
            q2, kw, (((1,), (1,)), ((), ())),
            preferred_element_type=jnp.float32)
        s0 = jax.lax.dot_general(
            q2, k0, (((1,), (1,)), ((), ())),
            preferred_element_type=jnp.float32)
        kw_pos = wstart + jax.lax.broadcasted_iota(
            jnp.int32, (_BQ2, _WIN), 1)
        mask_w = (kw_pos <= qpos) & ((kw_pos < _SINK)
                                     | ((qpos - kw_pos) < _LOCAL))
        k0_pos = jax.lax.broadcasted_iota(jnp.int32, (_BQ2, 256), 1)
        mask_0 = (k0_pos < _SINK) & (k0_pos < wstart)
        sw = jnp.where(mask_w, sw, _NEG)
        s0 = jnp.where(mask_0, s0, _NEG)
        m = jnp.maximum(jnp.max(sw, axis=1, keepdims=True),
                        jnp.max(s0, axis=1, keepdims=True))
        pw = jnp.exp((sw - m) * _SCALE)
        p0 = jnp.exp((s0 - m) * _SCALE)
        vw_ext = jnp.concatenate(
            [vw, jnp.ones((_WIN, _HD), jnp.bfloat16)], axis=1)
        v0_ext = jnp.concatenate(
            [v0, jnp.ones((256, _HD), jnp.bfloat16)], axis=1)
        o_ext = (jnp.dot(pw.astype(jnp.bfloat16), vw_ext,
                         preferred_element_type=jnp.float32)
                 + jnp.dot(p0.astype(jnp.bfloat16), v0_ext,
                           preferred_element_type=jnp.float32))
        m_s[...] = jnp.broadcast_to(m, (_BQ2, _HD))
        l_s[...] = o_ext[:, _HD:]
        o_s[...] = o_ext[:, :_HD]

    o2 = o_s[...] / l_s[...]
    o_ref[...] = jnp.concatenate(
        [o2[:_BQA, :], o2[_BQA:, :]], axis=1).astype(jnp.bfloat16)


def kernel(hidden_states, wqkv, wo):
    hs = hidden_states.reshape(_S, _HID)

    half = _HD // 2
    inv_freq = 1.0 / (_ROPE_THETA ** (
        jnp.arange(0, half, dtype=jnp.float32) / half))
    pos = jnp.arange(_S, dtype=jnp.float32)
    angles = pos[:, None] * inv_freq[None, :]
    ctab = jnp.concatenate([jnp.cos(angles)] * 2, axis=-1)
    stab = jnp.concatenate([jnp.sin(angles)] * 2, axis=-1)

    qkv = pl.pallas_call(
        _qkv_rope_kernel,
        grid=(_NQ,),
        in_specs=[
            pl.BlockSpec((_BQ, _HID), lambda i: (i, 0)),
            pl.BlockSpec((_HID, _QKV_W), lambda i: (0, 0)),
            pl.BlockSpec((_BQ, _HD), lambda i: (i, 0)),
            pl.BlockSpec((_BQ, _HD), lambda i: (i, 0)),
        ],
        out_specs=pl.BlockSpec((_BQ, _QKV_W), lambda i: (i, 0)),
        out_shape=jax.ShapeDtypeStruct((_S, _QKV_W), jnp.bfloat16),
        scratch_shapes=[pltpu.VMEM((_HID, _QKV_W), jnp.bfloat16)],
    )(hs, wqkv, ctab, stab)

    attn = pl.pallas_call(
        _attn_kernel,
        grid=(_NKV, _NQA),
        in_specs=[
            pl.BlockSpec((_BQA, 2 * _HD), lambda kh, qb: (qb, kh)),
            pl.BlockSpec((_S, _HD), lambda kh, qb: (0, _NH + kh)),
            pl.BlockSpec((_S, _HD), lambda kh, qb: (0, _NH + _NKV + kh)),
        ],
        out_specs=pl.BlockSpec((_BQA, 2 * _HD), lambda kh, qb: (qb, kh)),
        out_shape=jax.ShapeDtypeStruct((_S, _NH * _HD), jnp.bfloat16),
        scratch_shapes=[
            pltpu.VMEM((_BQ2, _HD), jnp.float32),
            pltpu.VMEM((_BQ2, _HD), jnp.float32),
            pltpu.VMEM((_BQ2, _HD), jnp.float32),
        ],
    )(qkv, qkv, qkv)

    out = pl.pallas_call(
        _out_proj_kernel,
        grid=(_NQ,),
        in_specs=[
            pl.BlockSpec((_BQ, _NH * _HD), lambda i: (i, 0)),
            pl.BlockSpec((_NH * _HD, _HID), lambda i: (0, 0)),
        ],
        out_specs=pl.BlockSpec((_BQ, _HID), lambda i: (i, 0)),
        out_shape=jax.ShapeDtypeStruct((_S, _HID), jnp.float32),
        scratch_shapes=[pltpu.VMEM((_NH * _HD, _HID), jnp.bfloat16)],
    )(attn, wo)

    return out.reshape(_B, _S, _HID)


# speedup vs baseline: 1.1727x; 1.0072x over previous
import jax
import jax.numpy as jnp
from jax.experimental import pallas as pl
from jax.experimental.pallas import tpu as pltpu

_B, _S, _HID = 1, 2048, 2048
_NH, _NKV, _HD = 16, 8, 128
_ROPE_THETA = 10000.0
_SINK, _LOCAL = 64, 256
_NRET = 8
_NRKV = _NRET // 2
_REP = _NH // _NKV
_QKV_W = (_NH + 2 * _NKV) * _HD
_QK_W = (_NH + _NKV) * _HD
_NQK = _NH + _NKV
_BQ = 256
_NQ = _S // _BQ
_BQA = 512
_BQ2 = 2 * _BQA
_NQA = _S // _BQA
_SCALE = _HD ** -0.5
_NEG = -1e30
_PART = 512
_NPART = _S // _PART


def _qkv_rope_kernel(x_ref, w_ref, ct_ref, st_ref, o_ref, wb_s):
    @pl.when(pl.program_id(0) == 0)
    def _cast():
        wb_s[...] = w_ref[...].astype(jnp.bfloat16)

    acc = jnp.dot(x_ref[...].astype(jnp.bfloat16), wb_s[...],
                  preferred_element_type=jnp.float32)
    qk = acc[:, :_QK_W]
    pieces = []
    for c in range(_NQK):
        x1 = qk[:, c * _HD: c * _HD + _HD // 2]
        x2 = qk[:, c * _HD + _HD // 2: (c + 1) * _HD]
        pieces += [-x2, x1]
    rot = jnp.concatenate(pieces, axis=1)
    ct = jnp.concatenate([ct_ref[...]] * _NQK, axis=1)
    st = jnp.concatenate([st_ref[...]] * _NQK, axis=1)
    roped = qk * ct + rot * st
    o_ref[...] = jnp.concatenate(
        [roped, acc[:, _QK_W:]], axis=1).astype(jnp.bfloat16)


def _out_proj_kernel(x_ref, w_ref, o_ref, wb_s):
    @pl.when(pl.program_id(0) == 0)
    def _cast():
        wb_s[...] = w_ref[...].astype(jnp.bfloat16)

    o_ref[...] = jnp.dot(x_ref[...], wb_s[...],
                         preferred_element_type=jnp.float32)


def _partial(q2, k_ref, v_ref, start, qpos, masked):
    k = k_ref[pl.ds(start, _PART), :]
    v = v_ref[pl.ds(start, _PART), :]
    s = jax.lax.dot_general(
        q2, k, (((1,), (1,)), ((), ())),
        preferred_element_type=jnp.float32)
    if masked:
        kpos = start + jax.lax.broadcasted_iota(jnp.int32, (_BQ2, _PART), 1)
        s = jnp.where(kpos <= qpos, s, _NEG)
    m = jnp.max(s, axis=1, keepdims=True)
    p = jnp.exp((s - m) * _SCALE)
    v_ext = jnp.concatenate(
        [v, jnp.ones((_PART, _HD), jnp.bfloat16)], axis=1)
    o_ext = jnp.dot(p.astype(jnp.bfloat16), v_ext,
                    preferred_element_type=jnp.float32)
    return m, o_ext[:, _HD:], o_ext[:, :_HD]


_WIN = 768


def _attn_kernel(q_ref, k_ref, v_ref, o_ref, m_s, l_s, o_s):
    kh = pl.program_id(0)
    qb = pl.program_id(1)
    qblk = q_ref[...]
    q2 = jnp.concatenate([qblk[:, :_HD], qblk[:, _HD:]], axis=0)
    qpos1 = qb * _BQA + jax.lax.broadcasted_iota(jnp.int32, (_BQA, 1), 0)
    qpos = jnp.concatenate([qpos1, qpos1], axis=0)

    @pl.when(kh < _NRKV)
    def _retrieval():
        def _init(masked):
            m0, l0, o0 = _partial(q2, k_ref, v_ref, 0, qpos, masked)
            m_s[...] = jnp.broadcast_to(m0, (_BQ2, _HD))
            l_s[...] = l0
            o_s[...] = o0

        def _accum(i, masked):
            mi, li, oi = _partial(q2, k_ref, v_ref, i * _PART, qpos, masked)
            m_prev = m_s[...]
            m_new = jnp.maximum(m_prev, mi)
            a = jnp.exp((m_prev - m_new) * _SCALE)
            b = jnp.exp((mi - m_new) * _SCALE)
            l_s[...] = a * l_s[...] + b * li
            o_s[...] = a * o_s[...] + b * oi
            m_s[...] = m_new

        _init(True)
        for i in range(1, _NPART):
            pl.when(qb >= i)(lambda i=i: _accum(i, True))

    @pl.when(kh >= _NRKV)
    def _streaming():
        wstart = jnp.maximum(qb * 2 - 1, 0) * 256
        kw = k_ref[pl.ds(wstart, _WIN), :]
        vw = v_ref[pl.ds(wstart, _WIN), :]
        k0 = k_ref[0:_SINK, :]
        v0 = v_ref[0:_SINK, :]
        sw = jax.lax.dot_general(
            q2, kw, (((1,), (1,)), ((), ())),
            preferred_element_type=jnp.float32)
        s0 = jax.lax.dot_general(
            q2, k0, (((1,), (1,)), ((), ())),
            preferred_element_type=jnp.float32)
        kw_pos = wstart + jax.lax.broadcasted_iota(
            jnp.int32, (_BQ2, _WIN), 1)
        mask_w = (kw_pos <= qpos) & ((kw_pos < _SINK)
                                     | ((qpos - kw_pos) < _LOCAL))
        k0_pos = jax.lax.broadcasted_iota(jnp.int32, (_BQ2, _SINK), 1)
        mask_0 = k0_pos < wstart
        sw = jnp.where(mask_w, sw, _NEG)
        s0 = jnp.where(mask_0, s0, _NEG)
        m = jnp.maximum(jnp.max(sw, axis=1, keepdims=True),
                        jnp.max(s0, axis=1, keepdims=True))
        pw = jnp.exp((sw - m) * _SCALE)
        p0 = jnp.exp((s0 - m) * _SCALE)
        vw_ext = jnp.concatenate(
            [vw, jnp.ones((_WIN, _HD), jnp.bfloat16)], axis=1)
        v0_ext = jnp.concatenate(
            [v0, jnp.ones((_SINK, _HD), jnp.bfloat16)], axis=1)
        o_ext = (jnp.dot(pw.astype(jnp.bfloat16), vw_ext,
                         preferred_element_type=jnp.float32)
                 + jnp.dot(p0.astype(jnp.bfloat16), v0_ext,
                           preferred_element_type=jnp.float32))
        m_s[...] = jnp.broadcast_to(m, (_BQ2, _HD))
        l_s[...] = o_ext[:, _HD:]
        o_s[...] = o_ext[:, :_HD]

    o2 = o_s[...] / l_s[...]
    o_ref[...] = jnp.concatenate(
        [o2[:_BQA, :], o2[_BQA:, :]], axis=1).astype(jnp.bfloat16)


def kernel(hidden_states, wqkv, wo):
    hs = hidden_states.reshape(_S, _HID)

    half = _HD // 2
    inv_freq = 1.0 / (_ROPE_THETA ** (
        jnp.arange(0, half, dtype=jnp.float32) / half))
    pos = jnp.arange(_S, dtype=jnp.float32)
    angles = pos[:, None] * inv_freq[None, :]
    ctab = jnp.concatenate([jnp.cos(angles)] * 2, axis=-1)
    stab = jnp.concatenate([jnp.sin(angles)] * 2, axis=-1)

    qkv = pl.pallas_call(
        _qkv_rope_kernel,
        grid=(_NQ,),
        in_specs=[
            pl.BlockSpec((_BQ, _HID), lambda i: (i, 0)),
            pl.BlockSpec((_HID, _QKV_W), lambda i: (0, 0)),
            pl.BlockSpec((_BQ, _HD), lambda i: (i, 0)),
            pl.BlockSpec((_BQ, _HD), lambda i: (i, 0)),
        ],
        out_specs=pl.BlockSpec((_BQ, _QKV_W), lambda i: (i, 0)),
        out_shape=jax.ShapeDtypeStruct((_S, _QKV_W), jnp.bfloat16),
        scratch_shapes=[pltpu.VMEM((_HID, _QKV_W), jnp.bfloat16)],
    )(hs, wqkv, ctab, stab)

    attn = pl.pallas_call(
        _attn_kernel,
        grid=(_NKV, _NQA),
        in_specs=[
            pl.BlockSpec((_BQA, 2 * _HD), lambda kh, qb: (qb, kh)),
            pl.BlockSpec((_S, _HD), lambda kh, qb: (0, _NH + kh)),
            pl.BlockSpec((_S, _HD), lambda kh, qb: (0, _NH + _NKV + kh)),
        ],
        out_specs=pl.BlockSpec((_BQA, 2 * _HD), lambda kh, qb: (qb, kh)),
        out_shape=jax.ShapeDtypeStruct((_S, _NH * _HD), jnp.bfloat16),
        scratch_shapes=[
            pltpu.VMEM((_BQ2, _HD), jnp.float32),
            pltpu.VMEM((_BQ2, _HD), jnp.float32),
            pltpu.VMEM((_BQ2, _HD), jnp.float32),
        ],
    )(qkv, qkv, qkv)

    out = pl.pallas_call(
        _out_proj_kernel,
        grid=(_NQ,),
        in_specs=[
            pl.BlockSpec((_BQ, _NH * _HD), lambda i: (i, 0)),
            pl.BlockSpec((_NH * _HD, _HID), lambda i: (0, 0)),
        ],
        out_specs=pl.BlockSpec((_BQ, _HID), lambda i: (i, 0)),
        out_shape=jax.ShapeDtypeStruct((_S, _HID), jnp.float32),
        scratch_shapes=[pltpu.VMEM((_NH * _HD, _HID), jnp.bfloat16)],
    )(attn, wo)

    return out.reshape(_B, _S, _HID)


# speedup vs baseline: 1.1748x; 1.0017x over previous
import jax
import jax.numpy as jnp
from jax.experimental import pallas as pl
from jax.experimental.pallas import tpu as pltpu

_B, _S, _HID = 1, 2048, 2048
_NH, _NKV, _HD = 16, 8, 128
_ROPE_THETA = 10000.0
_SINK, _LOCAL = 64, 256
_NRET = 8
_NRKV = _NRET // 2
_REP = _NH // _NKV
_QKV_W = (_NH + 2 * _NKV) * _HD
_QK_W = (_NH + _NKV) * _HD
_NQK = _NH + _NKV
_BQ = 256
_NQ = _S // _BQ
_BQA = 512
_BQ2 = 2 * _BQA
_NQA = _S // _BQA
_SCALE = _HD ** -0.5
_NEG = -1e30
_PART = 512
_NPART = _S // _PART


def _qkv_rope_kernel(x_ref, w_ref, ct_ref, st_ref, o_ref, wb_s):
    @pl.when(pl.program_id(0) == 0)
    def _cast():
        wb_s[...] = w_ref[...].astype(jnp.bfloat16)

    acc = jnp.dot(x_ref[...].astype(jnp.bfloat16), wb_s[...],
                  preferred_element_type=jnp.float32).astype(jnp.bfloat16)
    qk = acc[:, :_QK_W]
    pieces = []
    for c in range(_NQK):
        x1 = qk[:, c * _HD: c * _HD + _HD // 2]
        x2 = qk[:, c * _HD + _HD // 2: (c + 1) * _HD]
        pieces += [-x2, x1]
    rot = jnp.concatenate(pieces, axis=1)
    ct = jnp.concatenate([ct_ref[...]] * _NQK, axis=1)
    st = jnp.concatenate([st_ref[...]] * _NQK, axis=1)
    roped = qk * ct + rot * st
    o_ref[...] = jnp.concatenate([roped, acc[:, _QK_W:]], axis=1)


def _out_proj_kernel(x_ref, w_ref, o_ref, wb_s):
    @pl.when(pl.program_id(0) == 0)
    def _cast():
        wb_s[...] = w_ref[...].astype(jnp.bfloat16)

    o_ref[...] = jnp.dot(x_ref[...], wb_s[...],
                         preferred_element_type=jnp.float32)


def _partial(q2, k_ref, v_ref, start, qpos, masked):
    k = k_ref[pl.ds(start, _PART), :]
    v = v_ref[pl.ds(start, _PART), :]
    s = jax.lax.dot_general(
        q2, k, (((1,), (1,)), ((), ())),
        preferred_element_type=jnp.float32)
    if masked:
        kpos = start + jax.lax.broadcasted_iota(jnp.int32, (_BQ2, _PART), 1)
        s = jnp.where(kpos <= qpos, s, _NEG)
    m = jnp.max(s, axis=1, keepdims=True)
    p = jnp.exp((s - m) * _SCALE)
    v_ext = jnp.concatenate(
        [v, jnp.ones((_PART, _HD), jnp.bfloat16)], axis=1)
    o_ext = jnp.dot(p.astype(jnp.bfloat16), v_ext,
                    preferred_element_type=jnp.float32)
    return m, o_ext[:, _HD:], o_ext[:, :_HD]


_WIN = 768


def _attn_kernel(q_ref, k_ref, v_ref, o_ref, m_s, l_s, o_s):
    kh = pl.program_id(0)
    qb = pl.program_id(1)
    qblk = q_ref[...]
    q2 = jnp.concatenate([qblk[:, :_HD], qblk[:, _HD:]], axis=0)
    qpos1 = qb * _BQA + jax.lax.broadcasted_iota(jnp.int32, (_BQA, 1), 0)
    qpos = jnp.concatenate([qpos1, qpos1], axis=0)

    @pl.when(kh < _NRKV)
    def _retrieval():
        def _init(masked):
            m0, l0, o0 = _partial(q2, k_ref, v_ref, 0, qpos, masked)
            m_s[...] = jnp.broadcast_to(m0, (_BQ2, _HD))
            l_s[...] = l0
            o_s[...] = o0

        def _accum(i, masked):
            mi, li, oi = _partial(q2, k_ref, v_ref, i * _PART, qpos, masked)
            m_prev = m_s[...]
            m_new = jnp.maximum(m_prev, mi)
            a = jnp.exp((m_prev - m_new) * _SCALE)
            b = jnp.exp((mi - m_new) * _SCALE)
            l_s[...] = a * l_s[...] + b * li
            o_s[...] = a * o_s[...] + b * oi
            m_s[...] = m_new

        _init(True)
        for i in range(1, _NPART):
            pl.when(qb >= i)(lambda i=i: _accum(i, True))

    @pl.when(kh >= _NRKV)
    def _streaming():
        wstart = jnp.maximum(qb * 2 - 1, 0) * 256
        kw = k_ref[pl.ds(wstart, _WIN), :]
        vw = v_ref[pl.ds(wstart, _WIN), :]
        k0 = k_ref[0:_SINK, :]
        v0 = v_ref[0:_SINK, :]
        sw = jax.lax.dot_general(
            q2, kw, (((1,), (1,)), ((), ())),
            preferred_element_type=jnp.float32)
        s0 = jax.lax.dot_general(
            q2, k0, (((1,), (1,)), ((), ())),
            preferred_element_type=jnp.float32)
        kw_pos = wstart + jax.lax.broadcasted_iota(
            jnp.int32, (_BQ2, _WIN), 1)
        mask_w = (kw_pos <= qpos) & ((kw_pos < _SINK)
                                     | ((qpos - kw_pos) < _LOCAL))
        k0_pos = jax.lax.broadcasted_iota(jnp.int32, (_BQ2, _SINK), 1)
        mask_0 = k0_pos < wstart
        sw = jnp.where(mask_w, sw, _NEG)
        s0 = jnp.where(mask_0, s0, _NEG)
        m = jnp.maximum(jnp.max(sw, axis=1, keepdims=True),
                        jnp.max(s0, axis=1, keepdims=True))
        pw = jnp.exp((sw - m) * _SCALE)
        p0 = jnp.exp((s0 - m) * _SCALE)
        vw_ext = jnp.concatenate(
            [vw, jnp.ones((_WIN, _HD), jnp.bfloat16)], axis=1)
        v0_ext = jnp.concatenate(
            [v0, jnp.ones((_SINK, _HD), jnp.bfloat16)], axis=1)
        o_ext = (jnp.dot(pw.astype(jnp.bfloat16), vw_ext,
                         preferred_element_type=jnp.float32)
                 + jnp.dot(p0.astype(jnp.bfloat16), v0_ext,
                           preferred_element_type=jnp.float32))
        m_s[...] = jnp.broadcast_to(m, (_BQ2, _HD))
        l_s[...] = o_ext[:, _HD:]
        o_s[...] = o_ext[:, :_HD]

    o2 = o_s[...] / l_s[...]
    o_ref[...] = jnp.concatenate(
        [o2[:_BQA, :], o2[_BQA:, :]], axis=1).astype(jnp.bfloat16)


def kernel(hidden_states, wqkv, wo):
    hs = hidden_states.reshape(_S, _HID)

    half = _HD // 2
    inv_freq = 1.0 / (_ROPE_THETA ** (
        jnp.arange(0, half, dtype=jnp.float32) / half))
    pos = jnp.arange(_S, dtype=jnp.float32)
    angles = pos[:, None] * inv_freq[None, :]
    ctab = jnp.concatenate(
        [jnp.cos(angles)] * 2, axis=-1).astype(jnp.bfloat16)
    stab = jnp.concatenate(
        [jnp.sin(angles)] * 2, axis=-1).astype(jnp.bfloat16)

    qkv = pl.pallas_call(
        _qkv_rope_kernel,
        grid=(_NQ,),
        in_specs=[
            pl.BlockSpec((_BQ, _HID), lambda i: (i, 0)),
            pl.BlockSpec((_HID, _QKV_W), lambda i: (0, 0)),
            pl.BlockSpec((_BQ, _HD), lambda i: (i, 0)),
            pl.BlockSpec((_BQ, _HD), lambda i: (i, 0)),
        ],
        out_specs=pl.BlockSpec((_BQ, _QKV_W), lambda i: (i, 0)),
        out_shape=jax.ShapeDtypeStruct((_S, _QKV_W), jnp.bfloat16),
        scratch_shapes=[pltpu.VMEM((_HID, _QKV_W), jnp.bfloat16)],
    )(hs, wqkv, ctab, stab)

    attn = pl.pallas_call(
        _attn_kernel,
        grid=(_NKV, _NQA),
        in_specs=[
            pl.BlockSpec((_BQA, 2 * _HD), lambda kh, qb: (qb, kh)),
            pl.BlockSpec((_S, _HD), lambda kh, qb: (0, _NH + kh)),
            pl.BlockSpec((_S, _HD), lambda kh, qb: (0, _NH + _NKV + kh)),
        ],
        out_specs=pl.BlockSpec((_BQA, 2 * _HD), lambda kh, qb: (qb, kh)),
        out_shape=jax.ShapeDtypeStruct((_S, _NH * _HD), jnp.bfloat16),
        scratch_shapes=[
            pltpu.VMEM((_BQ2, _HD), jnp.float32),
            pltpu.VMEM((_BQ2, _HD), jnp.float32),
            pltpu.VMEM((_BQ2, _HD), jnp.float32),
        ],
    )(qkv, qkv, qkv)

    out = pl.pallas_call(
        _out_proj_kernel,
        grid=(_NQ,),
        in_specs=[
            pl.BlockSpec((_BQ, _NH * _HD), lambda i: (i, 0)),
            pl.BlockSpec((_NH * _HD, _HID), lambda i: (0, 0)),
        ],
        out_specs=pl.BlockSpec((_BQ, _HID), lambda i: (i, 0)),
        out_shape=jax.ShapeDtypeStruct((_S, _HID), jnp.float32),
        scratch_shapes=[pltpu.VMEM((_NH * _HD, _HID), jnp.bfloat16)],
    )(attn, wo)

    return out.reshape(_B, _S, _HID)
